# Initial kernel scaffold; baseline (speedup 1.0000x reference)
import jax
import jax.numpy as jnp
from jax.experimental import pallas as pl


def kernel(keypoints, points, bev_features, mlp_w1, mlp_w2, mlp_w3, fusion_w):
    raise NotImplementedError("write your pallas kernel here")


# speedup vs baseline: 7.9360x; 7.9360x over previous
import functools

import jax
import jax.numpy as jnp
from jax.experimental import pallas as pl

VOXEL_SIZE = 0.1
BEV_STRIDE = 8.0
PC_MIN_X = 0.0
PC_MIN_Y = -40.0
RADIUS_SQ = 4.0
NSAMPLE = 16

KT = 256
INTERPRET = False


def _vsa_body(kpT_ref, pT_ref, bev_ref, w1_ref, w2_ref, w3_ref, fwa_ref,
              fwb_ref, out_ref):
    kp = kpT_ref[0]
    p = pT_ref[0]
    n = p.shape[1]

    dx = kp[0][:, None] - p[0][None, :]
    dy = kp[1][:, None] - p[1][None, :]
    dz = kp[2][:, None] - p[2][None, :]
    d2 = dx * dx + dy * dy + dz * dz

    w1 = w1_ref[...]
    w2 = w2_ref[...]
    w3 = w3_ref[...]

    pw1 = jax.lax.dot_general(p, w1, (((0,), (0,)), ((), ())),
                              preferred_element_type=jnp.float32)
    kpw1 = jax.lax.dot_general(kp, w1, (((0,), (0,)), ((), ())),
                               preferred_element_type=jnp.float32)

    iota_n = jax.lax.broadcasted_iota(jnp.int32, (KT, n), 1)
    big = jnp.float32(3.0e38)

    def step(_, carry):
        d2c, pooled = carry
        m = jnp.min(d2c, axis=1, keepdims=True)
        is_min = d2c == m
        idx = jnp.min(jnp.where(is_min, iota_n, n), axis=1,
                      keepdims=True)
        onehot = iota_n == idx
        oh = onehot.astype(jnp.float32)
        z1 = jnp.dot(oh, pw1, preferred_element_type=jnp.float32) - kpw1
        h = jax.nn.relu(z1)
        h = jax.nn.relu(jnp.dot(h, w2, preferred_element_type=jnp.float32))
        h = jax.nn.relu(jnp.dot(h, w3, preferred_element_type=jnp.float32))
        within = m <= RADIUS_SQ
        pooled = jnp.maximum(pooled, jnp.where(within, h, 0.0))
        d2c = jnp.where(onehot, big, d2c)
        return d2c, pooled

    pooled0 = jnp.zeros((KT, w3.shape[1]), jnp.float32)
    _, pooled = jax.lax.fori_loop(0, NSAMPLE, step, (d2, pooled0))

    bev = bev_ref[0]
    fused = jnp.dot(bev, fwa_ref[...], preferred_element_type=jnp.float32)
    fused = fused + jnp.dot(pooled, fwb_ref[...],
                            preferred_element_type=jnp.float32)
    out_ref[0] = jax.nn.relu(fused)


def _bilinear_bev(keypoints, bev_features):
    B, C, H, W = bev_features.shape
    x = (keypoints[:, :, 0] - PC_MIN_X) / (VOXEL_SIZE * BEV_STRIDE)
    y = (keypoints[:, :, 1] - PC_MIN_Y) / (VOXEL_SIZE * BEV_STRIDE)
    x0 = jnp.clip(jnp.floor(x).astype(jnp.int32), 0, W - 1)
    x1 = jnp.clip(jnp.floor(x).astype(jnp.int32) + 1, 0, W - 1)
    y0 = jnp.clip(jnp.floor(y).astype(jnp.int32), 0, H - 1)
    y1 = jnp.clip(jnp.floor(y).astype(jnp.int32) + 1, 0, H - 1)
    ims = jnp.transpose(bev_features, (0, 2, 3, 1))
    def gather(im, yi, xi):
        return im[yi, xi]
    Ia = jax.vmap(gather)(ims, y0, x0)
    Ib = jax.vmap(gather)(ims, y1, x0)
    Ic = jax.vmap(gather)(ims, y0, x1)
    Id = jax.vmap(gather)(ims, y1, x1)
    xf0 = x0.astype(jnp.float32)
    xf1 = x1.astype(jnp.float32)
    yf0 = y0.astype(jnp.float32)
    yf1 = y1.astype(jnp.float32)
    wa = (xf1 - x) * (yf1 - y)
    wb = (xf1 - x) * (y - yf0)
    wc = (x - xf0) * (yf1 - y)
    wd = (x - xf0) * (y - yf0)
    return (Ia * wa[..., None] + Ib * wb[..., None] + Ic * wc[..., None]
            + Id * wd[..., None])


def kernel(keypoints, points, bev_features, mlp_w1, mlp_w2, mlp_w3, fusion_w):
    B, K, _ = keypoints.shape
    N = points.shape[1]
    C = bev_features.shape[1]

    point_bev = _bilinear_bev(keypoints, bev_features)

    kpT = jnp.transpose(keypoints, (0, 2, 1))
    pT = jnp.transpose(points, (0, 2, 1))
    fwa = fusion_w[:C]
    fwb = fusion_w[C:]

    nkb = K // KT
    grid = (B * nkb,)

    out = pl.pallas_call(
        _vsa_body,
        grid=grid,
        in_specs=[
            pl.BlockSpec((1, 3, KT), lambda i: (i // nkb, 0, i % nkb)),
            pl.BlockSpec((1, 3, N), lambda i: (i // nkb, 0, 0)),
            pl.BlockSpec((1, KT, C), lambda i: (i // nkb, i % nkb, 0)),
            pl.BlockSpec((3, 8), lambda i: (0, 0)),
            pl.BlockSpec((8, 16), lambda i: (0, 0)),
            pl.BlockSpec((16, 32), lambda i: (0, 0)),
            pl.BlockSpec((C, 128), lambda i: (0, 0)),
            pl.BlockSpec((32, 128), lambda i: (0, 0)),
        ],
        out_specs=pl.BlockSpec((1, KT, 128), lambda i: (i // nkb, i % nkb, 0)),
        out_shape=jax.ShapeDtypeStruct((B, K, 128), jnp.float32),
        interpret=INTERPRET,
    )(kpT, pT, point_bev, mlp_w1, mlp_w2, mlp_w3, fwa, fwb)
    return out
